# baseline (device time: 222091 ns/iter reference)
import jax
import jax.numpy as jnp
from jax import lax
from jax.experimental import pallas as pl
from jax.experimental.pallas import tpu as pltpu

N_DEV = 8
CAP = 640
CAP_W = CAP // 8


def _a2av_pallas(xb3, order, cnt_row):
    n_win, _, d_model = xb3.shape
    t_loc = n_win * 8
    send_win = N_DEV * CAP_W
    idx_rows = t_loc // 128

    def body(x_ref, order_ref, cnt_ref, out_ref,
             send_ref, recv_ref, cnt_mat_ref, idx_vmem,
             cnt_smem, idx_smem,
             dsend, drecv, csend, crecv, loc_sem):
        me = lax.axis_index("i")
        iota8 = lax.broadcasted_iota(jnp.int32, (8, d_model), 0)

        barrier_sem = pltpu.get_barrier_semaphore()
        for dd in range(1, N_DEV):
            t = lax.rem(me + dd, N_DEV)
            pl.semaphore_signal(barrier_sem, inc=1, device_id=(t,),
                                device_id_type=pl.DeviceIdType.MESH)
        pl.semaphore_wait(barrier_sem, N_DEV - 1)

        crdmas = []
        for dd in range(1, N_DEV):
            t = lax.rem(me + dd, N_DEV)
            c = pltpu.make_async_remote_copy(
                src_ref=cnt_ref,
                dst_ref=cnt_mat_ref.at[pl.ds(dd - 1, 1)],
                send_sem=csend.at[dd - 1],
                recv_sem=crecv.at[dd - 1],
                device_id=(t,),
                device_id_type=pl.DeviceIdType.MESH,
            )
            c.start()
            crdmas.append(c)

        cp0 = pltpu.make_async_copy(
            cnt_ref, cnt_smem.at[pl.ds(N_DEV - 1, 1)], loc_sem)
        cp0.start()
        cp0.wait()

        my_offs = []
        acc = jnp.int32(0)
        for r in range(N_DEV):
            my_offs.append(acc)
            acc = acc + cnt_smem[N_DEV - 1, r]

        for j in range(N_DEV):
            t_j = lax.rem(me + 1 + j, N_DEV)
            cnt_j = cnt_smem[N_DEV - 1, t_j]
            off_j = jnp.int32(0)
            for r in range(N_DEV):
                off_j = jnp.where(t_j == r, my_offs[r], off_j)
            kmax = jnp.maximum(cnt_j - 1, 0)

            def gather_it(itb, carry, off_j=off_j, kmax=kmax, j=j):
                acc = jnp.zeros((8, d_model), jnp.bfloat16)
                for u in range(8):
                    kk = itb * 8 + u
                    sj = jnp.minimum(off_j + jnp.minimum(kk, kmax), t_loc - 1)
                    r = order_ref[sj]
                    w = x_ref[lax.div(r, 8)]
                    sh = lax.rem(u - lax.rem(r, 8) + 8, 8)
                    acc = jnp.where(iota8 == u, pltpu.roll(w, sh, 0), acc)
                send_ref[j * CAP_W + itb] = acc
                return carry

            lax.fori_loop(0, CAP_W, gather_it, 0)

        self_cp = pltpu.make_async_copy(
            send_ref.at[pl.ds((N_DEV - 1) * CAP_W, CAP_W)],
            recv_ref.at[pl.ds((N_DEV - 1) * CAP_W, CAP_W)],
            loc_sem,
        )
        self_cp.start()

        rdmas = []
        for dd in range(1, N_DEV):
            t = lax.rem(me + dd, N_DEV)
            rdma = pltpu.make_async_remote_copy(
                src_ref=send_ref.at[pl.ds((dd - 1) * CAP_W, CAP_W)],
                dst_ref=recv_ref.at[pl.ds((dd - 1) * CAP_W, CAP_W)],
                send_sem=dsend.at[dd - 1],
                recv_sem=drecv.at[dd - 1],
                device_id=(t,),
                device_id_type=pl.DeviceIdType.MESH,
            )
            rdma.start()
            rdmas.append(rdma)

        for c in crdmas:
            c.wait()
        self_cp.wait()
        cp1 = pltpu.make_async_copy(
            cnt_mat_ref, cnt_smem.at[pl.ds(0, N_DEV - 1)], loc_sem)
        cp1.start()
        cp1.wait()

        slots, cums = [], []
        cum = jnp.int32(0)
        for s in range(N_DEV):
            slot = lax.rem(me + (N_DEV - 1) - s, N_DEV)
            slots.append(slot)
            cum = cum + cnt_smem[slot, me]
            cums.append(cum)

        j_vec = (lax.broadcasted_iota(jnp.int32, (idx_rows, 128), 0) * 128
                 + lax.broadcasted_iota(jnp.int32, (idx_rows, 128), 1))
        seg = jnp.zeros((idx_rows, 128), jnp.int32)
        for s in range(N_DEV - 1):
            seg = seg + (j_vec >= cums[s]).astype(jnp.int32)
        slot_v = jnp.zeros((idx_rows, 128), jnp.int32)
        off_v = jnp.zeros((idx_rows, 128), jnp.int32)
        for s in range(N_DEV):
            m = (seg == s).astype(jnp.int32)
            slot_v = slot_v + m * slots[s]
            off_v = off_v + m * (cums[s] - cnt_smem[slots[s], me])
        idx_vmem[:, :] = slot_v * CAP + (j_vec - off_v)

        cp3 = pltpu.make_async_copy(idx_vmem, idx_smem, loc_sem)
        cp3.start()
        cp3.wait()

        for rdma in rdmas:
            rdma.wait()

        def compact_it(it, carry):
            acc = jnp.zeros((8, d_model), jnp.bfloat16)
            for u in range(8):
                j = it * 8 + u
                f = idx_smem[lax.div(j, 128), lax.rem(j, 128)]
                w = recv_ref[lax.div(f, 8)]
                sh = lax.rem(u - lax.rem(f, 8) + 8, 8)
                acc = jnp.where(iota8 == u, pltpu.roll(w, sh, 0), acc)
            out_ref[it] = acc
            return carry

        lax.fori_loop(0, n_win, compact_it, 0)

    return pl.pallas_call(
        body,
        out_shape=jax.ShapeDtypeStruct((n_win, 8, d_model), jnp.bfloat16),
        in_specs=[
            pl.BlockSpec(memory_space=pltpu.VMEM),
            pl.BlockSpec(memory_space=pltpu.SMEM),
            pl.BlockSpec(memory_space=pltpu.VMEM),
        ],
        out_specs=pl.BlockSpec(memory_space=pltpu.VMEM),
        scratch_shapes=[
            pltpu.VMEM((send_win, 8, d_model), jnp.bfloat16),
            pltpu.VMEM((send_win, 8, d_model), jnp.bfloat16),
            pltpu.VMEM((N_DEV - 1, 128), jnp.int32),
            pltpu.VMEM((idx_rows, 128), jnp.int32),
            pltpu.SMEM((N_DEV, 128), jnp.int32),
            pltpu.SMEM((idx_rows, 128), jnp.int32),
            pltpu.SemaphoreType.DMA((N_DEV - 1,)),
            pltpu.SemaphoreType.DMA((N_DEV - 1,)),
            pltpu.SemaphoreType.DMA((N_DEV - 1,)),
            pltpu.SemaphoreType.DMA((N_DEV - 1,)),
            pltpu.SemaphoreType.DMA,
        ],
        compiler_params=pltpu.CompilerParams(collective_id=0),
    )(xb3, order, cnt_row)


def kernel(x, dest):
    t_loc, d_model = x.shape

    xb = x.astype(jnp.bfloat16)
    order = jnp.argsort(dest, stable=True).astype(jnp.int32)
    oh = (dest[:, None] == jnp.arange(N_DEV)[None, :]).astype(jnp.int32)
    cnt = oh.sum(axis=0).astype(jnp.int32)
    cnt_row = jnp.pad(cnt, (0, 128 - N_DEV)).reshape(1, 128)

    out3 = _a2av_pallas(xb.reshape(t_loc // 8, 8, d_model), order, cnt_row)
    return out3.reshape(t_loc, d_model)
